# baseline (speedup 1.0000x reference)
import functools

import jax
import jax.numpy as jnp
from jax import lax
from jax.experimental import pallas as pl
from jax.experimental.pallas import tpu as pltpu
from jax.experimental.pallas import tpu_sc as plsc

N_DIMS = 3
N_FIELDS = 64
RES = 64
GRID = RES + 1
N_POINTS = 131072
N_CELLS = GRID * GRID * GRID

NC = 2
NS = 16
L = 16
NW = NC * NS
PTS_PER_W = N_POINTS // NW
C = 32
NCHUNK = PTS_PER_W // C
NIDX = 8 * C
SEG = 128
NSEG = NIDX // SEG


def _sc_kernel(xt_hbm, tab_hbm, out_hbm, xv, idxv, rows, outv,
               gsem0, gsem1, osem0, osem1):
    wid = lax.axis_index("s") * NC + lax.axis_index("c")
    base = wid * PTS_PER_W
    gsems = (gsem0, gsem1)
    osems = (osem0, osem1)

    for d in range(N_DIMS):
        pltpu.sync_copy(
            xt_hbm.at[pl.ds(d * N_POINTS + base, PTS_PER_W)],
            xv.at[pl.ds(d * PTS_PER_W, PTS_PER_W)],
        )

    def coords(t, g):
        fl, fr = [], []
        for d in range(N_DIMS):
            td = xv[pl.ds(d * PTS_PER_W + t * C + g * L, L)] * float(RES)
            fld = td.astype(jnp.int32)
            fl.append(fld)
            fr.append(td - fld.astype(jnp.float32))
        return fl, fr

    def build_and_fire(t, b):
        for g in range(C // L):
            fl, _ = coords(t, g)
            flat = (fl[0] * GRID + fl[1]) * GRID + fl[2]
            for c in range(8):
                dx, dy, dz = (c >> 2) & 1, (c >> 1) & 1, c & 1
                cidx = flat + (dx * GRID * GRID + dy * GRID + dz)
                j = c * C + g * L
                idxv[b, j // SEG, pl.ds(j % SEG, L)] = cidx
        for s in range(NSEG):
            pltpu.async_copy(
                tab_hbm.at[idxv.at[b, s]],
                rows.at[b, pl.ds(s * SEG, SEG)],
                gsems[b],
            )

    def drain_gather(b):
        for s in range(NSEG):
            pltpu.make_async_copy(
                tab_hbm.at[idxv.at[b, s]],
                rows.at[b, pl.ds(s * SEG, SEG)],
                gsems[b],
            ).wait()

    def compute(t, b):
        for g in range(C // L):
            _, fr = coords(t, g)
            for i in range(L):
                p = g * L + i
                fx = fr[0][i]
                fy = fr[1][i]
                fz = fr[2][i]
                for r in range(N_FIELDS // L):
                    sl = pl.ds(r * L, L)
                    v = [rows[b, c * C + p, sl] for c in range(8)]
                    a00 = v[0] + (v[1] - v[0]) * fz
                    a01 = v[2] + (v[3] - v[2]) * fz
                    a10 = v[4] + (v[5] - v[4]) * fz
                    a11 = v[6] + (v[7] - v[6]) * fz
                    b0 = a00 + (a01 - a00) * fy
                    b1 = a10 + (a11 - a10) * fy
                    outv[b, p // 8, p % 8, pl.ds(r * L, L)] = (
                        b0 + (b1 - b0) * fx
                    )

    def fire_out(t, b):
        pltpu.async_copy(
            outv.at[b], out_hbm.at[pl.ds((base + t * C) // 8, C // 8)], osems[b]
        )

    def drain_out(t, b):
        pltpu.make_async_copy(
            outv.at[b], out_hbm.at[pl.ds((base + t * C) // 8, C // 8)], osems[b]
        ).wait()

    build_and_fire(0, 0)
    build_and_fire(1, 1)

    def pair_body(t2, carry):
        t = 2 * t2
        for b in range(2):
            tb = t + b

            @pl.when(t2 > 0)
            def _():
                drain_out(tb - 2, b)

            drain_gather(b)
            compute(tb, b)
            fire_out(tb, b)

            @pl.when(t2 < NCHUNK // 2 - 1)
            def _():
                build_and_fire(tb + 2, b)

        return carry

    lax.fori_loop(0, NCHUNK // 2, pair_body, 0)
    drain_out(NCHUNK - 2, 0)
    drain_out(NCHUNK - 1, 1)


@jax.jit
def kernel(x, values):
    xt = x.T.reshape(-1)
    tab = jnp.concatenate(
        [
            values[0:21].reshape(-1),
            values[21:43].reshape(-1),
            values[43:65].reshape(-1),
        ]
    ).reshape(N_CELLS, N_FIELDS)
    run = functools.partial(
        pl.kernel,
        out_type=jax.ShapeDtypeStruct((N_POINTS // 8, 8, 128), jnp.float32),
        mesh=plsc.VectorSubcoreMesh(core_axis_name="c", subcore_axis_name="s"),
        compiler_params=pltpu.CompilerParams(use_tc_tiling_on_sc=False),
        scratch_types=[
            pltpu.VMEM((PTS_PER_W * N_DIMS,), jnp.float32),
            pltpu.VMEM((2, NSEG, SEG), jnp.int32),
            pltpu.VMEM((2, NIDX, N_FIELDS), jnp.float32),
            pltpu.VMEM((2, C // 8, 8, 128), jnp.float32),
            pltpu.SemaphoreType.DMA,
            pltpu.SemaphoreType.DMA,
            pltpu.SemaphoreType.DMA,
            pltpu.SemaphoreType.DMA,
        ],
    )(_sc_kernel)
    out = run(xt, tab)
    return out.reshape(N_POINTS, 128)[:, :N_FIELDS]


# speedup vs baseline: 2.2371x; 2.2371x over previous
import functools

import jax
import jax.numpy as jnp
from jax import lax
from jax.experimental import pallas as pl
from jax.experimental.pallas import tpu as pltpu
from jax.experimental.pallas import tpu_sc as plsc

N_DIMS = 3
N_FIELDS = 64
RES = 64
GRID = RES + 1
N_POINTS = 131072
N_CELLS = GRID * GRID * GRID

NC = 2
NS = 16
L = 16
NW = NC * NS
PTS_PER_W = N_POINTS // NW
C = 16
NCHUNK = PTS_PER_W // C
NIDX = 8 * C
SEG = 128
NSEG = NIDX // SEG
NBUF = 4


def _sc_kernel(xt_hbm, tab_hbm, out_hbm, xv, idxv, rows, outv, *sems):
    gsems = sems[:NBUF]
    osems = sems[NBUF:]
    wid = lax.axis_index("s") * NC + lax.axis_index("c")
    base = wid * PTS_PER_W

    for d in range(N_DIMS):
        pltpu.sync_copy(
            xt_hbm.at[pl.ds(d * N_POINTS + base, PTS_PER_W)],
            xv.at[pl.ds(d * PTS_PER_W, PTS_PER_W)],
        )

    def coords(t, g):
        fl, fr = [], []
        for d in range(N_DIMS):
            td = xv[pl.ds(d * PTS_PER_W + t * C + g * L, L)] * float(RES)
            fld = td.astype(jnp.int32)
            fl.append(fld)
            fr.append(td - fld.astype(jnp.float32))
        return fl, fr

    def build_and_fire(t, b):
        for g in range(C // L):
            fl, _ = coords(t, g)
            flat = (fl[0] * GRID + fl[1]) * GRID + fl[2]
            for c in range(8):
                dx, dy, dz = (c >> 2) & 1, (c >> 1) & 1, c & 1
                cidx = flat + (dx * GRID * GRID + dy * GRID + dz)
                j = c * C + g * L
                idxv[b, j // SEG, pl.ds(j % SEG, L)] = cidx
        for s in range(NSEG):
            pltpu.async_copy(
                tab_hbm.at[idxv.at[b, s]],
                rows.at[b, pl.ds(s * SEG, SEG)],
                gsems[b],
            )

    def drain_gather(b):
        for s in range(NSEG):
            pltpu.make_async_copy(
                tab_hbm.at[idxv.at[b, s]],
                rows.at[b, pl.ds(s * SEG, SEG)],
                gsems[b],
            ).wait()

    def compute(t, b):
        for g in range(C // L):
            _, fr = coords(t, g)
            for i in range(L):
                p = g * L + i
                fx = fr[0][i]
                fy = fr[1][i]
                fz = fr[2][i]
                for r in range(N_FIELDS // L):
                    sl = pl.ds(r * L, L)
                    v = [rows[b, c * C + p, sl] for c in range(8)]
                    a00 = v[0] + (v[1] - v[0]) * fz
                    a01 = v[2] + (v[3] - v[2]) * fz
                    a10 = v[4] + (v[5] - v[4]) * fz
                    a11 = v[6] + (v[7] - v[6]) * fz
                    b0 = a00 + (a01 - a00) * fy
                    b1 = a10 + (a11 - a10) * fy
                    outv[b, p // 8, p % 8, pl.ds(r * L, L)] = (
                        b0 + (b1 - b0) * fx
                    )

    def fire_out(t, b):
        pltpu.async_copy(
            outv.at[b], out_hbm.at[pl.ds((base + t * C) // 8, C // 8)], osems[b]
        )

    def drain_out(t, b):
        pltpu.make_async_copy(
            outv.at[b], out_hbm.at[pl.ds((base + t * C) // 8, C // 8)], osems[b]
        ).wait()

    for b in range(NBUF):
        build_and_fire(b, b)

    def ring_body(tq, carry):
        t = NBUF * tq
        for b in range(NBUF):
            tb = t + b

            @pl.when(tq > 0)
            def _():
                drain_out(tb - NBUF, b)

            drain_gather(b)
            compute(tb, b)
            fire_out(tb, b)

            @pl.when(tq < NCHUNK // NBUF - 1)
            def _():
                build_and_fire(tb + NBUF, b)

        return carry

    lax.fori_loop(0, NCHUNK // NBUF, ring_body, 0)
    for b in range(NBUF):
        drain_out(NCHUNK - NBUF + b, b)


@jax.jit
def kernel(x, values):
    xt = x.T.reshape(-1)
    tab = values.reshape(N_CELLS, N_FIELDS)
    run = functools.partial(
        pl.kernel,
        out_type=jax.ShapeDtypeStruct((N_POINTS // 8, 8, 128), jnp.float32),
        mesh=plsc.VectorSubcoreMesh(core_axis_name="c", subcore_axis_name="s"),
        compiler_params=pltpu.CompilerParams(use_tc_tiling_on_sc=False),
        scratch_types=[
            pltpu.VMEM((N_DIMS * PTS_PER_W,), jnp.float32),
            pltpu.VMEM((NBUF, NSEG, SEG), jnp.int32),
            pltpu.VMEM((NBUF, NIDX, N_FIELDS), jnp.float32),
            pltpu.VMEM((NBUF, C // 8, 8, 128), jnp.float32),
        ] + [pltpu.SemaphoreType.DMA] * (2 * NBUF),
    )(_sc_kernel)
    out = run(xt, tab)
    return out.reshape(N_POINTS, 128)[:, :N_FIELDS]
